# baseline (speedup 1.0000x reference)
import functools

import jax
import jax.numpy as jnp
from jax import lax
from jax.experimental import pallas as pl
from jax.experimental.pallas import tpu as pltpu
from jax.experimental.pallas import tpu_sc as plsc

_B = 64
_N = 32768
_D = 128
_D1 = 16
_HID = 64
_K = 16
_C = 128
_NCHUNK = _N // _C
_BLK = 8192
_NBLK = _N // _BLK
_CPB = _BLK // _C


def _phase1_body(query_ref, wq_ref, keys_ref, scores_ref, cid_ref, fid_ref,
                 q_s, cm_s):
    i = pl.program_id(0)

    @pl.when(i == 0)
    def _():
        q_s[...] = lax.dot_general(
            query_ref[...], wq_ref[...],
            (((1,), (1,)), ((), ())), preferred_element_type=jnp.float32)

    slabs, cms = [], []
    for c in range(_CPB):
        s_c = lax.dot_general(
            q_s[...], keys_ref[pl.ds(c * _C, _C), :],
            (((1,), (1,)), ((), ())), preferred_element_type=jnp.float32)
        slabs.append(s_c)
        cms.append(jnp.max(s_c, axis=1, keepdims=True))
    scores_ref[...] = jnp.concatenate(slabs, axis=0)
    cm = jnp.concatenate(cms, axis=1)
    pad = jnp.full((_B, 128 - _CPB), -jnp.inf, jnp.float32)
    cm_s[:, pl.ds(i * 128, 128)] = jnp.concatenate([cm, pad], axis=1)

    @pl.when(i == _NBLK - 1)
    def _():
        M = cm_s[...].reshape(_B, _NBLK, 128)[:, :, :_CPB].reshape(_B, _NCHUNK)
        pcol = lax.broadcasted_iota(jnp.int32, (_B, _NCHUNK), 1)
        ois = []
        for _ in range(_K):
            m = jnp.max(M, axis=1, keepdims=True)
            eq = M == m
            p = jnp.min(jnp.where(eq, pcol, _NCHUNK), axis=1, keepdims=True)
            ois.append(p)
            M = jnp.where(pcol == p, -jnp.inf, M)
        pad_i = jnp.zeros((_B, 128 - _K), jnp.int32)
        new_i = jnp.concatenate(ois + [pad_i], axis=1)
        cid_ref[...] = new_i
        row = lax.broadcasted_iota(jnp.int32, (_B, 128), 0)
        fid_ref[...] = new_i * _B + row


def _phase1_call(query, keys, W_q):
    return pl.pallas_call(
        _phase1_body,
        grid=(_NBLK,),
        in_specs=[
            pl.BlockSpec((_B, _D), lambda i: (0, 0)),
            pl.BlockSpec((_D, _D), lambda i: (0, 0)),
            pl.BlockSpec((_BLK, _D), lambda i: (i, 0)),
        ],
        out_specs=[
            pl.BlockSpec((_CPB * _B, _C), lambda i: (i, 0)),
            pl.BlockSpec((_B, 128), lambda i: (0, 0)),
            pl.BlockSpec((_B, 128), lambda i: (0, 0)),
        ],
        out_shape=[
            jax.ShapeDtypeStruct((_NCHUNK * _B, _C), jnp.float32),
            jax.ShapeDtypeStruct((_B, 128), jnp.int32),
            jax.ShapeDtypeStruct((_B, 128), jnp.int32),
        ],
        scratch_shapes=[
            pltpu.VMEM((_B, _D), jnp.float32),
            pltpu.VMEM((_B, _NBLK * 128), jnp.float32),
        ],
        compiler_params=pltpu.CompilerParams(
            dimension_semantics=("arbitrary",),
        ),
    )(query, W_q, keys)


def _sc_gather(table, idx_flat, nrows):
    info = plsc.get_sparse_core_info()
    nw = info.num_cores * info.num_subcores
    bpw = nrows // nw
    mesh = plsc.VectorSubcoreMesh(core_axis_name="c", subcore_axis_name="s")

    @functools.partial(
        pl.kernel,
        mesh=mesh,
        out_type=jax.ShapeDtypeStruct((nrows, table.shape[1]), table.dtype),
        scratch_types=[
            pltpu.VMEM((bpw,), jnp.int32),
            pltpu.VMEM((bpw, table.shape[1]), table.dtype),
            pltpu.SemaphoreType.DMA,
        ],
    )
    def k(table_hbm, idx_hbm, out_hbm, idx_v, rows_v, sem):
        wid = lax.axis_index("s") * info.num_cores + lax.axis_index("c")
        base = wid * bpw
        pltpu.sync_copy(idx_hbm.at[pl.ds(base, bpw)], idx_v)
        pltpu.async_copy(table_hbm.at[idx_v], rows_v, sem).wait()
        pltpu.sync_copy(rows_v, out_hbm.at[pl.ds(base, bpw)])

    return k(table, idx_flat)


def _phase2_body(cand_ref, cid_ref, vals_ref, vfid_ref):
    s3 = cand_ref[...]
    cid = cid_ref[...][:, :_K]
    gidx3 = cid[:, :, None] * _C + lax.broadcasted_iota(
        jnp.int32, (_B, _K, _C), 2)
    big = jnp.int32(1 << 30)
    ngrp = 4
    gs = [s3[:, g * 4:(g + 1) * 4, :] for g in range(ngrp)]
    gi = [gidx3[:, g * 4:(g + 1) * 4, :] for g in range(ngrp)]
    chains_v = [[] for _ in range(ngrp)]
    chains_g = [[] for _ in range(ngrp)]
    for _ in range(_K):
        for g in range(ngrp):
            sg, ig = gs[g], gi[g]
            m2 = jnp.max(sg, axis=2, keepdims=True)
            m = jnp.max(m2, axis=1, keepdims=True)
            eq = sg == m
            g2 = jnp.min(jnp.where(eq, ig, big), axis=2, keepdims=True)
            gmin = jnp.min(g2, axis=1, keepdims=True)
            chains_v[g].append(m.reshape(_B, 1))
            chains_g[g].append(gmin.reshape(_B, 1))
            gs[g] = jnp.where(ig == gmin, -jnp.inf, sg)
    allv = jnp.concatenate(sum(chains_v, []), axis=1)
    alli = jnp.concatenate(sum(chains_g, []), axis=1)
    ovs, ogs = [], []
    for _ in range(_K):
        m = jnp.max(allv, axis=1, keepdims=True)
        g = jnp.min(jnp.where(allv == m, alli, big), axis=1, keepdims=True)
        ovs.append(m)
        ogs.append(g)
        allv = jnp.where(alli == g, -jnp.inf, allv)
    pad_v = jnp.zeros((_B, 128 - _K), jnp.float32)
    pad_i = jnp.zeros((_B, 128 - _K), jnp.int32)
    vals_ref[...] = jnp.concatenate(ovs + [pad_v], axis=1)
    vfid_ref[...] = jnp.concatenate(ogs + [pad_i], axis=1)


def _phase2_call(cand3, cid_pad):
    return pl.pallas_call(
        _phase2_body,
        out_shape=[
            jax.ShapeDtypeStruct((_B, 128), jnp.float32),
            jax.ShapeDtypeStruct((_B, 128), jnp.int32),
        ],
    )(cand3, cid_pad)


def _epi_body(vtop_ref, sv_ref, pf_ref, a1v_ref, a1p_ref, b1_ref, a2_ref,
              b2_ref, g_ref, be_ref, out_ref):
    vtop = vtop_ref[...]
    h = lax.dot_general(vtop, a1v_ref[...], (((1,), (1,)), ((), ())),
                        preferred_element_type=jnp.float32)
    pfc = lax.dot_general(pf_ref[...], a1p_ref[...], (((1,), (1,)), ((), ())),
                          preferred_element_type=jnp.float32)
    h = h.reshape(_B, _K, _HID) + pfc[:, None, :] + b1_ref[...][None, :, :]
    h = jnp.maximum(h, 0.0).reshape(_B * _K, _HID)
    h2 = lax.dot_general(h, a2_ref[...], (((1,), (1,)), ((), ())),
                         preferred_element_type=jnp.float32) + b2_ref[...]
    mu = jnp.mean(h2, axis=1, keepdims=True)
    var = jnp.mean((h2 - mu) ** 2, axis=1, keepdims=True)
    ln = (h2 - mu) * lax.rsqrt(var + 1e-5) * g_ref[...] + be_ref[...]
    adapted = vtop + ln
    a = sv_ref[...][:, :_K] * (1.0 / (_D ** 0.5))
    m = jnp.max(a, axis=1, keepdims=True)
    e = jnp.exp(a - m)
    w = e / jnp.sum(e, axis=1, keepdims=True)
    out_ref[...] = jnp.sum(adapted.reshape(_B, _K, _D) * w[:, :, None], axis=1)


def _epi_call(vtop, sv, pf, a1v, a1p, b1, a2, b2, g, be):
    return pl.pallas_call(
        _epi_body,
        out_shape=jax.ShapeDtypeStruct((_B, _D), jnp.float32),
    )(vtop, sv, pf, a1v, a1p, b1, a2, b2, g, be)


def kernel(query, keys, values, param_feats, top_k, W_q, A1, b1, A2, b2, gamma, beta):
    scores, cid_pad, fid_pad = _phase1_call(query, keys, W_q)
    fid_flat = fid_pad[:, :_K].reshape(_B * _K)
    cand = _sc_gather(scores, fid_flat, _B * _K)
    tv, vfid = _phase2_call(cand.reshape(_B, _K, _C), cid_pad)
    idx_flat = vfid[:, :_K].reshape(_B * _K)
    vtop = _sc_gather(values, idx_flat, _B * _K)
    out = _epi_call(
        vtop, tv, param_feats,
        A1[:, :_D], A1[:, _D:], b1.reshape(1, _HID),
        A2, b2.reshape(1, _D),
        gamma.reshape(1, _D), beta.reshape(1, _D))
    return out


# speedup vs baseline: 1.3440x; 1.3440x over previous
import functools

import jax
import jax.numpy as jnp
from jax import lax
from jax.experimental import pallas as pl
from jax.experimental.pallas import tpu as pltpu
from jax.experimental.pallas import tpu_sc as plsc

_B = 64
_N = 32768
_D = 128
_D1 = 16
_HID = 64
_K = 16
_C = 128
_NCHUNK = _N // _C
_BLK = 8192
_NBLK = _N // _BLK
_CPB = _BLK // _C


def _phase1_body(query_ref, wq_ref, keys_ref, scores_ref, cid_ref, fid_ref,
                 q_s, cm_s):
    i = pl.program_id(0)

    @pl.when(i == 0)
    def _():
        q_s[...] = lax.dot_general(
            query_ref[...], wq_ref[...],
            (((1,), (1,)), ((), ())), preferred_element_type=jnp.float32)

    slabs, cms = [], []
    for c in range(_CPB):
        s_c = lax.dot_general(
            q_s[...], keys_ref[pl.ds(c * _C, _C), :],
            (((1,), (1,)), ((), ())), preferred_element_type=jnp.float32)
        slabs.append(s_c)
        cms.append(jnp.max(s_c, axis=1, keepdims=True))
    scores_ref[...] = jnp.concatenate(slabs, axis=0)
    cm = jnp.concatenate(cms, axis=1)
    pad = jnp.full((_B, 128 - _CPB), -jnp.inf, jnp.float32)
    cm_s[:, pl.ds(i * 128, 128)] = jnp.concatenate([cm, pad], axis=1)

    @pl.when(i == _NBLK - 1)
    def _():
        M = cm_s[...].reshape(_B, _NBLK, 128)[:, :, :_CPB].reshape(_B, _NCHUNK)
        pcol = lax.broadcasted_iota(jnp.int32, (_B, _NCHUNK), 1)
        ois = []
        for _ in range(_K):
            m = jnp.max(M, axis=1, keepdims=True)
            eq = M == m
            p = jnp.min(jnp.where(eq, pcol, _NCHUNK), axis=1, keepdims=True)
            ois.append(p)
            M = jnp.where(pcol == p, -jnp.inf, M)
        pad_i = jnp.zeros((_B, 128 - _K), jnp.int32)
        new_i = jnp.concatenate(ois + [pad_i], axis=1)
        cid_ref[...] = new_i
        row = lax.broadcasted_iota(jnp.int32, (_B, 128), 0)
        fid_ref[...] = new_i * _B + row


def _phase1_call(query, keys, W_q):
    return pl.pallas_call(
        _phase1_body,
        grid=(_NBLK,),
        in_specs=[
            pl.BlockSpec((_B, _D), lambda i: (0, 0)),
            pl.BlockSpec((_D, _D), lambda i: (0, 0)),
            pl.BlockSpec((_BLK, _D), lambda i: (i, 0)),
        ],
        out_specs=[
            pl.BlockSpec((_CPB * _B, _C), lambda i: (i, 0)),
            pl.BlockSpec((_B, 128), lambda i: (0, 0)),
            pl.BlockSpec((_B, 128), lambda i: (0, 0)),
        ],
        out_shape=[
            jax.ShapeDtypeStruct((_NCHUNK * _B, _C), jnp.float32),
            jax.ShapeDtypeStruct((_B, 128), jnp.int32),
            jax.ShapeDtypeStruct((_B, 128), jnp.int32),
        ],
        scratch_shapes=[
            pltpu.VMEM((_B, _D), jnp.float32),
            pltpu.VMEM((_B, _NBLK * 128), jnp.float32),
        ],
        compiler_params=pltpu.CompilerParams(
            dimension_semantics=("arbitrary",),
        ),
    )(query, W_q, keys)


def _sc_gather(table, idx_flat, nrows):
    info = plsc.get_sparse_core_info()
    nw = info.num_cores * info.num_subcores
    bpw = nrows // nw
    mesh = plsc.VectorSubcoreMesh(core_axis_name="c", subcore_axis_name="s")

    @functools.partial(
        pl.kernel,
        mesh=mesh,
        out_type=jax.ShapeDtypeStruct((nrows, table.shape[1]), table.dtype),
        scratch_types=[
            pltpu.VMEM((bpw,), jnp.int32),
            pltpu.VMEM((bpw, table.shape[1]), table.dtype),
            pltpu.SemaphoreType.DMA,
        ],
    )
    def k(table_hbm, idx_hbm, out_hbm, idx_v, rows_v, sem):
        wid = lax.axis_index("s") * info.num_cores + lax.axis_index("c")
        base = wid * bpw
        pltpu.sync_copy(idx_hbm.at[pl.ds(base, bpw)], idx_v)
        pltpu.async_copy(table_hbm.at[idx_v], rows_v, sem).wait()
        pltpu.sync_copy(rows_v, out_hbm.at[pl.ds(base, bpw)])

    return k(table, idx_flat)


def _phase2_body(cand_ref, cid_ref, vals_ref, vfid_ref):
    s3 = cand_ref[...]
    cid = cid_ref[...][:, :_K]
    gidx3 = cid[:, :, None] * _C + lax.broadcasted_iota(
        jnp.int32, (_B, _K, _C), 2)
    big = jnp.int32(1 << 30)
    ovs, ogs = [], []
    for _ in range(_K):
        m2 = jnp.max(s3, axis=2, keepdims=True)
        m = jnp.max(m2, axis=1, keepdims=True)
        eq = s3 == m
        g2 = jnp.min(jnp.where(eq, gidx3, big), axis=2, keepdims=True)
        g = jnp.min(g2, axis=1, keepdims=True)
        ovs.append(m.reshape(_B, 1))
        ogs.append(g.reshape(_B, 1))
        s3 = jnp.where(gidx3 == g, -jnp.inf, s3)
    pad_v = jnp.zeros((_B, 128 - _K), jnp.float32)
    pad_i = jnp.zeros((_B, 128 - _K), jnp.int32)
    vals_ref[...] = jnp.concatenate(ovs + [pad_v], axis=1)
    vfid_ref[...] = jnp.concatenate(ogs + [pad_i], axis=1)


def _phase2_call(cand3, cid_pad):
    return pl.pallas_call(
        _phase2_body,
        out_shape=[
            jax.ShapeDtypeStruct((_B, 128), jnp.float32),
            jax.ShapeDtypeStruct((_B, 128), jnp.int32),
        ],
    )(cand3, cid_pad)


def _epi_body(vtop_ref, sv_ref, pf_ref, a1v_ref, a1p_ref, b1_ref, a2_ref,
              b2_ref, g_ref, be_ref, out_ref):
    vtop = vtop_ref[...]
    h = lax.dot_general(vtop, a1v_ref[...], (((1,), (1,)), ((), ())),
                        preferred_element_type=jnp.float32)
    pfc = lax.dot_general(pf_ref[...], a1p_ref[...], (((1,), (1,)), ((), ())),
                          preferred_element_type=jnp.float32)
    h = h.reshape(_B, _K, _HID) + pfc[:, None, :] + b1_ref[...][None, :, :]
    h = jnp.maximum(h, 0.0).reshape(_B * _K, _HID)
    h2 = lax.dot_general(h, a2_ref[...], (((1,), (1,)), ((), ())),
                         preferred_element_type=jnp.float32) + b2_ref[...]
    mu = jnp.mean(h2, axis=1, keepdims=True)
    var = jnp.mean((h2 - mu) ** 2, axis=1, keepdims=True)
    ln = (h2 - mu) * lax.rsqrt(var + 1e-5) * g_ref[...] + be_ref[...]
    adapted = vtop + ln
    a = sv_ref[...][:, :_K] * (1.0 / (_D ** 0.5))
    m = jnp.max(a, axis=1, keepdims=True)
    e = jnp.exp(a - m)
    w = e / jnp.sum(e, axis=1, keepdims=True)
    out_ref[...] = jnp.sum(adapted.reshape(_B, _K, _D) * w[:, :, None], axis=1)


def _epi_call(vtop, sv, pf, a1v, a1p, b1, a2, b2, g, be):
    return pl.pallas_call(
        _epi_body,
        out_shape=jax.ShapeDtypeStruct((_B, _D), jnp.float32),
    )(vtop, sv, pf, a1v, a1p, b1, a2, b2, g, be)


def kernel(query, keys, values, param_feats, top_k, W_q, A1, b1, A2, b2, gamma, beta):
    scores, cid_pad, fid_pad = _phase1_call(query, keys, W_q)
    fid_flat = fid_pad[:, :_K].reshape(_B * _K)
    cand = _sc_gather(scores, fid_flat, _B * _K)
    tv, vfid = _phase2_call(cand.reshape(_B, _K, _C), cid_pad)
    idx_flat = vfid[:, :_K].reshape(_B * _K)
    vtop = _sc_gather(values, idx_flat, _B * _K)
    out = _epi_call(
        vtop, tv, param_feats,
        A1[:, :_D], A1[:, _D:], b1.reshape(1, _HID),
        A2, b2.reshape(1, _D),
        gamma.reshape(1, _D), beta.reshape(1, _D))
    return out


# speedup vs baseline: 1.3558x; 1.0088x over previous
import functools

import jax
import jax.numpy as jnp
from jax import lax
from jax.experimental import pallas as pl
from jax.experimental.pallas import tpu as pltpu
from jax.experimental.pallas import tpu_sc as plsc

_B = 64
_N = 32768
_D = 128
_D1 = 16
_HID = 64
_K = 16
_C = 128
_NCHUNK = _N // _C
_BLK = 8192
_NBLK = _N // _BLK
_CPB = _BLK // _C


def _phase1_body(query_ref, wq_ref, keys_ref, scores_ref, cid_ref, fid_ref,
                 q_s, cm_s):
    i = pl.program_id(0)

    @pl.when(i == 0)
    def _():
        q_s[...] = lax.dot_general(
            query_ref[...], wq_ref[...],
            (((1,), (1,)), ((), ())), preferred_element_type=jnp.float32)

    cms = []
    for c in range(_CPB):
        s_c = lax.dot_general(
            q_s[...], keys_ref[pl.ds(c * _C, _C), :],
            (((1,), (1,)), ((), ())), preferred_element_type=jnp.float32)
        scores_ref[pl.ds(c * _B, _B), :] = s_c
        cms.append(jnp.max(s_c, axis=1, keepdims=True))
    cm = jnp.concatenate(cms, axis=1)
    pad = jnp.full((_B, 128 - _CPB), -jnp.inf, jnp.float32)
    cm_s[:, pl.ds(i * 128, 128)] = jnp.concatenate([cm, pad], axis=1)

    @pl.when(i == _NBLK - 1)
    def _():
        M = cm_s[...].reshape(_B, _NBLK, 128)[:, :, :_CPB].reshape(_B, _NCHUNK)
        pcol = lax.broadcasted_iota(jnp.int32, (_B, _NCHUNK), 1)
        ois = []
        for _ in range(_K):
            m = jnp.max(M, axis=1, keepdims=True)
            eq = M == m
            p = jnp.min(jnp.where(eq, pcol, _NCHUNK), axis=1, keepdims=True)
            ois.append(p)
            M = jnp.where(pcol == p, -jnp.inf, M)
        pad_i = jnp.zeros((_B, 128 - _K), jnp.int32)
        new_i = jnp.concatenate(ois + [pad_i], axis=1)
        cid_ref[...] = new_i
        row = lax.broadcasted_iota(jnp.int32, (_B, 128), 0)
        fid_ref[...] = new_i * _B + row


def _phase1_call(query, keys, W_q):
    return pl.pallas_call(
        _phase1_body,
        grid=(_NBLK,),
        in_specs=[
            pl.BlockSpec((_B, _D), lambda i: (0, 0)),
            pl.BlockSpec((_D, _D), lambda i: (0, 0)),
            pl.BlockSpec((_BLK, _D), lambda i: (i, 0)),
        ],
        out_specs=[
            pl.BlockSpec((_CPB * _B, _C), lambda i: (i, 0)),
            pl.BlockSpec((_B, 128), lambda i: (0, 0)),
            pl.BlockSpec((_B, 128), lambda i: (0, 0)),
        ],
        out_shape=[
            jax.ShapeDtypeStruct((_NCHUNK * _B, _C), jnp.float32),
            jax.ShapeDtypeStruct((_B, 128), jnp.int32),
            jax.ShapeDtypeStruct((_B, 128), jnp.int32),
        ],
        scratch_shapes=[
            pltpu.VMEM((_B, _D), jnp.float32),
            pltpu.VMEM((_B, _NBLK * 128), jnp.float32),
        ],
        compiler_params=pltpu.CompilerParams(
            dimension_semantics=("arbitrary",),
        ),
    )(query, W_q, keys)


def _sc_gather(table, idx_flat, nrows):
    info = plsc.get_sparse_core_info()
    nw = info.num_cores * info.num_subcores
    bpw = nrows // nw
    mesh = plsc.VectorSubcoreMesh(core_axis_name="c", subcore_axis_name="s")

    @functools.partial(
        pl.kernel,
        mesh=mesh,
        out_type=jax.ShapeDtypeStruct((nrows, table.shape[1]), table.dtype),
        scratch_types=[
            pltpu.VMEM((bpw,), jnp.int32),
            pltpu.VMEM((bpw, table.shape[1]), table.dtype),
            pltpu.SemaphoreType.DMA,
        ],
    )
    def k(table_hbm, idx_hbm, out_hbm, idx_v, rows_v, sem):
        wid = lax.axis_index("s") * info.num_cores + lax.axis_index("c")
        base = wid * bpw
        pltpu.sync_copy(idx_hbm.at[pl.ds(base, bpw)], idx_v)
        pltpu.async_copy(table_hbm.at[idx_v], rows_v, sem).wait()
        pltpu.sync_copy(rows_v, out_hbm.at[pl.ds(base, bpw)])

    return k(table, idx_flat)


def _phase2_body(cand_ref, cid_ref, vals_ref, vfid_ref):
    s3 = cand_ref[...]
    cid = cid_ref[...][:, :_K]
    gidx3 = cid[:, :, None] * _C + lax.broadcasted_iota(
        jnp.int32, (_B, _K, _C), 2)
    big = jnp.int32(1 << 30)
    ovs, ogs = [], []
    for _ in range(_K):
        m2 = jnp.max(s3, axis=2, keepdims=True)
        m = jnp.max(m2, axis=1, keepdims=True)
        eq = s3 == m
        g2 = jnp.min(jnp.where(eq, gidx3, big), axis=2, keepdims=True)
        g = jnp.min(g2, axis=1, keepdims=True)
        ovs.append(m.reshape(_B, 1))
        ogs.append(g.reshape(_B, 1))
        s3 = jnp.where(gidx3 == g, -jnp.inf, s3)
    pad_v = jnp.zeros((_B, 128 - _K), jnp.float32)
    pad_i = jnp.zeros((_B, 128 - _K), jnp.int32)
    vals_ref[...] = jnp.concatenate(ovs + [pad_v], axis=1)
    vfid_ref[...] = jnp.concatenate(ogs + [pad_i], axis=1)


def _phase2_call(cand3, cid_pad):
    return pl.pallas_call(
        _phase2_body,
        out_shape=[
            jax.ShapeDtypeStruct((_B, 128), jnp.float32),
            jax.ShapeDtypeStruct((_B, 128), jnp.int32),
        ],
    )(cand3, cid_pad)


def _epi_body(vtop_ref, sv_ref, pf_ref, a1v_ref, a1p_ref, b1_ref, a2_ref,
              b2_ref, g_ref, be_ref, out_ref):
    vtop = vtop_ref[...]
    h = jnp.dot(vtop, a1v_ref[...], preferred_element_type=jnp.float32)
    pfc = jnp.dot(pf_ref[...], a1p_ref[...], preferred_element_type=jnp.float32)
    h = h.reshape(_B, _K, _HID) + pfc[:, None, :] + b1_ref[...][None, :, :]
    h = jnp.maximum(h, 0.0).reshape(_B * _K, _HID)
    h2 = jnp.dot(h, a2_ref[...], preferred_element_type=jnp.float32) + b2_ref[...]
    mu = jnp.mean(h2, axis=1, keepdims=True)
    var = jnp.mean((h2 - mu) ** 2, axis=1, keepdims=True)
    ln = (h2 - mu) * lax.rsqrt(var + 1e-5) * g_ref[...] + be_ref[...]
    adapted = vtop + ln
    a = sv_ref[...][:, :_K] * (1.0 / (_D ** 0.5))
    m = jnp.max(a, axis=1, keepdims=True)
    e = jnp.exp(a - m)
    w = e / jnp.sum(e, axis=1, keepdims=True)
    out_ref[...] = jnp.sum(adapted.reshape(_B, _K, _D) * w[:, :, None], axis=1)


def _epi_call(vtop, sv, pf, a1v, a1p, b1, a2, b2, g, be):
    return pl.pallas_call(
        _epi_body,
        out_shape=jax.ShapeDtypeStruct((_B, _D), jnp.float32),
    )(vtop, sv, pf, a1v, a1p, b1, a2, b2, g, be)


def kernel(query, keys, values, param_feats, top_k, W_q, A1, b1, A2, b2, gamma, beta):
    scores, cid_pad, fid_pad = _phase1_call(query, keys, W_q)
    fid_flat = fid_pad[:, :_K].reshape(_B * _K)
    cand = _sc_gather(scores, fid_flat, _B * _K)
    tv, vfid = _phase2_call(cand.reshape(_B, _K, _C), cid_pad)
    idx_flat = vfid[:, :_K].reshape(_B * _K)
    vtop = _sc_gather(values, idx_flat, _B * _K)
    out = _epi_call(
        vtop, tv, param_feats,
        A1[:, :_D].T, A1[:, _D:].T, b1.reshape(1, _HID),
        A2.T, b2.reshape(1, _D),
        gamma.reshape(1, _D), beta.reshape(1, _D))
    return out


# speedup vs baseline: 1.3611x; 1.0039x over previous
import functools

import jax
import jax.numpy as jnp
from jax import lax
from jax.experimental import pallas as pl
from jax.experimental.pallas import tpu as pltpu
from jax.experimental.pallas import tpu_sc as plsc

_B = 64
_N = 32768
_D = 128
_D1 = 16
_HID = 64
_K = 16
_C = 128
_NCHUNK = _N // _C
_BLK = 16384
_NBLK = _N // _BLK
_CPB = _BLK // _C


def _phase1_body(query_ref, wq_ref, keys_ref, scores_ref, cid_ref, fid_ref,
                 q_s, cm_s):
    i = pl.program_id(0)

    @pl.when(i == 0)
    def _():
        q_s[...] = lax.dot_general(
            query_ref[...], wq_ref[...],
            (((1,), (1,)), ((), ())), preferred_element_type=jnp.float32)

    cms = []
    for c in range(_CPB):
        s_c = lax.dot_general(
            q_s[...], keys_ref[pl.ds(c * _C, _C), :],
            (((1,), (1,)), ((), ())), preferred_element_type=jnp.float32)
        scores_ref[pl.ds(c * _B, _B), :] = s_c
        cms.append(jnp.max(s_c, axis=1, keepdims=True))
    if _CPB < 128:
        cms.append(jnp.full((_B, 128 - _CPB), -jnp.inf, jnp.float32))
    cm_s[:, pl.ds(i * 128, 128)] = jnp.concatenate(cms, axis=1)

    @pl.when(i == _NBLK - 1)
    def _():
        M = cm_s[...].reshape(_B, _NBLK, 128)[:, :, :_CPB].reshape(_B, _NCHUNK)
        pcol = lax.broadcasted_iota(jnp.int32, (_B, _NCHUNK), 1)
        ois = []
        for _ in range(_K):
            m = jnp.max(M, axis=1, keepdims=True)
            eq = M == m
            p = jnp.min(jnp.where(eq, pcol, _NCHUNK), axis=1, keepdims=True)
            ois.append(p)
            M = jnp.where(pcol == p, -jnp.inf, M)
        pad_i = jnp.zeros((_B, 128 - _K), jnp.int32)
        new_i = jnp.concatenate(ois + [pad_i], axis=1)
        cid_ref[...] = new_i
        row = lax.broadcasted_iota(jnp.int32, (_B, 128), 0)
        fid_ref[...] = new_i * _B + row


def _phase1_call(query, keys, W_q):
    return pl.pallas_call(
        _phase1_body,
        grid=(_NBLK,),
        in_specs=[
            pl.BlockSpec((_B, _D), lambda i: (0, 0)),
            pl.BlockSpec((_D, _D), lambda i: (0, 0)),
            pl.BlockSpec((_BLK, _D), lambda i: (i, 0)),
        ],
        out_specs=[
            pl.BlockSpec((_CPB * _B, _C), lambda i: (i, 0)),
            pl.BlockSpec((_B, 128), lambda i: (0, 0)),
            pl.BlockSpec((_B, 128), lambda i: (0, 0)),
        ],
        out_shape=[
            jax.ShapeDtypeStruct((_NCHUNK * _B, _C), jnp.float32),
            jax.ShapeDtypeStruct((_B, 128), jnp.int32),
            jax.ShapeDtypeStruct((_B, 128), jnp.int32),
        ],
        scratch_shapes=[
            pltpu.VMEM((_B, _D), jnp.float32),
            pltpu.VMEM((_B, _NBLK * 128), jnp.float32),
        ],
        compiler_params=pltpu.CompilerParams(
            dimension_semantics=("arbitrary",),
        ),
    )(query, W_q, keys)


def _sc_gather(table, idx_flat, nrows):
    info = plsc.get_sparse_core_info()
    nw = info.num_cores * info.num_subcores
    bpw = nrows // nw
    mesh = plsc.VectorSubcoreMesh(core_axis_name="c", subcore_axis_name="s")

    @functools.partial(
        pl.kernel,
        mesh=mesh,
        out_type=jax.ShapeDtypeStruct((nrows, table.shape[1]), table.dtype),
        scratch_types=[
            pltpu.VMEM((bpw,), jnp.int32),
            pltpu.VMEM((bpw, table.shape[1]), table.dtype),
            pltpu.SemaphoreType.DMA,
        ],
    )
    def k(table_hbm, idx_hbm, out_hbm, idx_v, rows_v, sem):
        wid = lax.axis_index("s") * info.num_cores + lax.axis_index("c")
        base = wid * bpw
        pltpu.sync_copy(idx_hbm.at[pl.ds(base, bpw)], idx_v)
        pltpu.async_copy(table_hbm.at[idx_v], rows_v, sem).wait()
        pltpu.sync_copy(rows_v, out_hbm.at[pl.ds(base, bpw)])

    return k(table, idx_flat)


def _phase2_body(cand_ref, cid_ref, vals_ref, vfid_ref):
    s3 = cand_ref[...]
    cid = cid_ref[...][:, :_K]
    gidx3 = cid[:, :, None] * _C + lax.broadcasted_iota(
        jnp.int32, (_B, _K, _C), 2)
    big = jnp.int32(1 << 30)
    ovs, ogs = [], []
    for _ in range(_K):
        m2 = jnp.max(s3, axis=2, keepdims=True)
        m = jnp.max(m2, axis=1, keepdims=True)
        eq = s3 == m
        g2 = jnp.min(jnp.where(eq, gidx3, big), axis=2, keepdims=True)
        g = jnp.min(g2, axis=1, keepdims=True)
        ovs.append(m.reshape(_B, 1))
        ogs.append(g.reshape(_B, 1))
        s3 = jnp.where(gidx3 == g, -jnp.inf, s3)
    pad_v = jnp.zeros((_B, 128 - _K), jnp.float32)
    pad_i = jnp.zeros((_B, 128 - _K), jnp.int32)
    vals_ref[...] = jnp.concatenate(ovs + [pad_v], axis=1)
    vfid_ref[...] = jnp.concatenate(ogs + [pad_i], axis=1)


def _phase2_call(cand3, cid_pad):
    return pl.pallas_call(
        _phase2_body,
        out_shape=[
            jax.ShapeDtypeStruct((_B, 128), jnp.float32),
            jax.ShapeDtypeStruct((_B, 128), jnp.int32),
        ],
    )(cand3, cid_pad)


def _epi_body(vtop_ref, sv_ref, pf_ref, a1v_ref, a1p_ref, b1_ref, a2_ref,
              b2_ref, g_ref, be_ref, out_ref):
    vtop = vtop_ref[...]
    h = jnp.dot(vtop, a1v_ref[...], preferred_element_type=jnp.float32)
    pfc = jnp.dot(pf_ref[...], a1p_ref[...], preferred_element_type=jnp.float32)
    h = h.reshape(_B, _K, _HID) + pfc[:, None, :] + b1_ref[...][None, :, :]
    h = jnp.maximum(h, 0.0).reshape(_B * _K, _HID)
    h2 = jnp.dot(h, a2_ref[...], preferred_element_type=jnp.float32) + b2_ref[...]
    mu = jnp.mean(h2, axis=1, keepdims=True)
    var = jnp.mean((h2 - mu) ** 2, axis=1, keepdims=True)
    ln = (h2 - mu) * lax.rsqrt(var + 1e-5) * g_ref[...] + be_ref[...]
    adapted = vtop + ln
    a = sv_ref[...][:, :_K] * (1.0 / (_D ** 0.5))
    m = jnp.max(a, axis=1, keepdims=True)
    e = jnp.exp(a - m)
    w = e / jnp.sum(e, axis=1, keepdims=True)
    out_ref[...] = jnp.sum(adapted.reshape(_B, _K, _D) * w[:, :, None], axis=1)


def _epi_call(vtop, sv, pf, a1v, a1p, b1, a2, b2, g, be):
    return pl.pallas_call(
        _epi_body,
        out_shape=jax.ShapeDtypeStruct((_B, _D), jnp.float32),
    )(vtop, sv, pf, a1v, a1p, b1, a2, b2, g, be)


def kernel(query, keys, values, param_feats, top_k, W_q, A1, b1, A2, b2, gamma, beta):
    scores, cid_pad, fid_pad = _phase1_call(query, keys, W_q)
    fid_flat = fid_pad[:, :_K].reshape(_B * _K)
    cand = _sc_gather(scores, fid_flat, _B * _K)
    tv, vfid = _phase2_call(cand.reshape(_B, _K, _C), cid_pad)
    idx_flat = vfid[:, :_K].reshape(_B * _K)
    vtop = _sc_gather(values, idx_flat, _B * _K)
    out = _epi_call(
        vtop, tv, param_feats,
        A1[:, :_D].T, A1[:, _D:].T, b1.reshape(1, _HID),
        A2.T, b2.reshape(1, _D),
        gamma.reshape(1, _D), beta.reshape(1, _D))
    return out
